# baseline (speedup 1.0000x reference)
import jax
import jax.numpy as jnp
from jax import lax
from jax.experimental import pallas as pl
from jax.experimental.pallas import tpu as pltpu
from jax.experimental.pallas import tpu_sc as plsc

N = 10000
NPAD = 10240
E = 320000
D = 128
DH = D // 2
P = 64
PH = P // 2

NCORES = 2
NSUB = 16
NTILES = NCORES * NSUB

IDXW = 80
NROWS = E // IDXW
RPT = NROWS // NSUB
K = 5
CHUNKS = RPT // K
PAIRS = CHUNKS // 2
CE = K * IDXW
ROWS_PT = NPAD // NSUB
B = 2048

_MESH = plsc.VectorSubcoreMesh(core_axis_name="c", subcore_axis_name="s")
_SC_PARAMS = pltpu.CompilerParams(needs_layout_passes=False,
                                  use_tc_tiling_on_sc=False)


def _agg_call(with_cnt):
    out_type = [
        jax.ShapeDtypeStruct((NPAD, DH), jnp.float32),
        jax.ShapeDtypeStruct((NPAD, DH), jnp.float32),
    ]
    if with_cnt:
        out_type.append(jax.ShapeDtypeStruct((NSUB, NPAD), jnp.float32))
    scratch = [
        pltpu.VMEM((2, K, IDXW), jnp.int32),
        pltpu.VMEM((2, K, IDXW), jnp.int32),
        pltpu.VMEM((2, K, IDXW, DH), jnp.float32),
        pltpu.VMEM_SHARED((NPAD, DH), jnp.float32),
        pltpu.SemaphoreType.DMA,
        pltpu.SemaphoreType.DMA,
        pltpu.SemaphoreType.DMA,
        pltpu.SemaphoreType.DMA,
        pltpu.SemaphoreType.DMA,
        pltpu.SemaphoreType.DMA,
    ]
    if with_cnt:
        scratch.append(pltpu.VMEM((NPAD,), jnp.float32))

    def body(h_hbm, sidx_hbm, didx_hbm, zrows_hbm, *rest):
        if with_cnt:
            (out0, out1, out_cnt, sidx_v, didx_v, rows_v, acc_sh,
             isem0, isem1, gsem0, gsem1, ssem0, ssem1, cnt_v) = rest
        else:
            (out0, out1, sidx_v, didx_v, rows_v, acc_sh,
             isem0, isem1, gsem0, gsem1, ssem0, ssem1) = rest
        c = lax.axis_index("c")
        s = lax.axis_index("s")
        pltpu.sync_copy(zrows_hbm, acc_sh.at[pl.ds(s * ROWS_PT, ROWS_PT)])
        if with_cnt:
            def zero_cnt(i, carry):
                cnt_v[pl.ds(i * 16, 16)] = jnp.zeros((16,), jnp.float32)
                return carry
            lax.fori_loop(0, NPAD // 16, zero_cnt, 0)
        plsc.subcore_barrier()
        h_half = h_hbm.at[c]
        s_rows = sidx_hbm.at[s]
        d_rows = didx_hbm.at[s]
        isems = (isem0, isem1)
        gsems = (gsem0, gsem1)
        ssems = (ssem0, ssem1)

        def fire_i(chunk, b):
            pltpu.async_copy(s_rows.at[pl.ds(chunk * K, K)], sidx_v.at[b],
                             isems[b])
            pltpu.async_copy(d_rows.at[pl.ds(chunk * K, K)], didx_v.at[b],
                             isems[b])

        def wait_i(b):
            pltpu.make_async_copy(s_rows.at[pl.ds(0, K)], sidx_v.at[b],
                                  isems[b]).wait()
            pltpu.make_async_copy(d_rows.at[pl.ds(0, K)], didx_v.at[b],
                                  isems[b]).wait()

        def fire_g(b):
            for j in range(K):
                pltpu.async_copy(h_half.at[sidx_v.at[b, j]],
                                 rows_v.at[b, j], gsems[b])

        def wait_g(b):
            for j in range(K):
                pltpu.make_async_copy(h_half.at[sidx_v.at[b, 0]],
                                      rows_v.at[b, j], gsems[b]).wait()

        def fire_s(b):
            for j in range(K):
                pltpu.async_copy(rows_v.at[b, j],
                                 acc_sh.at[didx_v.at[b, j]],
                                 ssems[b], add=True)

        def wait_s(b):
            for j in range(K):
                pltpu.make_async_copy(rows_v.at[b, j],
                                      acc_sh.at[didx_v.at[b, 0]],
                                      ssems[b]).wait()

        def do_cnt(b):
            if with_cnt:
                ones16 = jnp.ones((16,), jnp.float32)
                for j in range(K):
                    for jj in range(IDXW // 16):
                        idx16 = didx_v[b, j, pl.ds(jj * 16, 16)]
                        plsc.addupdate_scatter(cnt_v, [idx16], ones16)

        fire_i(0, 0)

        def pair(i, carry):
            c0 = 2 * i
            c1 = 2 * i + 1

            @pl.when(i > 0)
            def _():
                wait_s(1)
            fire_i(c1, 1)
            wait_i(0)
            fire_g(0)
            do_cnt(0)
            wait_g(0)
            fire_s(0)
            wait_i(1)
            fire_g(1)
            do_cnt(1)
            wait_s(0)

            @pl.when(i < PAIRS - 1)
            def _():
                fire_i(c0 + 2, 0)
            wait_g(1)
            fire_s(1)
            return carry

        lax.fori_loop(0, PAIRS, pair, 0)
        wait_s(1)

        if with_cnt:
            @pl.when(c == 0)
            def _():
                pltpu.sync_copy(cnt_v, out_cnt.at[s])

        plsc.subcore_barrier()
        sl = pl.ds(s * ROWS_PT, ROWS_PT)

        @pl.when(c == 0)
        def _():
            pltpu.sync_copy(acc_sh.at[sl], out0.at[sl])

        @pl.when(c == 1)
        def _():
            pltpu.sync_copy(acc_sh.at[sl], out1.at[sl])

    return pl.kernel(body, out_type=out_type, mesh=_MESH,
                     scratch_types=scratch, compiler_params=_SC_PARAMS)


def _mbuild_call():
    out_type = [
        jax.ShapeDtypeStruct((NPAD, PH), jnp.bfloat16),
        jax.ShapeDtypeStruct((NPAD, PH), jnp.bfloat16),
    ]
    scratch = [
        pltpu.VMEM((RPT, IDXW), jnp.int32),
        pltpu.VMEM((RPT, IDXW), jnp.int32),
        pltpu.VMEM((2, CE, PH), jnp.bfloat16),
        pltpu.VMEM_SHARED((NPAD, PH), jnp.bfloat16),
        pltpu.SemaphoreType.DMA,
        pltpu.SemaphoreType.DMA,
        pltpu.SemaphoreType.DMA,
        pltpu.SemaphoreType.DMA,
    ]

    def body(pmt_hbm, sidx_hbm, didx_hbm, zrows_hbm,
             out0, out1, sidx_v, didx_v, rows_v, m_sh,
             lsem0, lsem1, ssem0, ssem1):
        c = lax.axis_index("c")
        s = lax.axis_index("s")
        pltpu.sync_copy(zrows_hbm, m_sh.at[pl.ds(s * ROWS_PT, ROWS_PT)])
        pltpu.sync_copy(sidx_hbm.at[s], sidx_v)
        pltpu.sync_copy(didx_hbm.at[s], didx_v)
        plsc.subcore_barrier()
        pmt_half = pmt_hbm.at[c]
        e_base = s * (E // NSUB)
        lsems = (lsem0, lsem1)
        ssems = (ssem0, ssem1)

        def fire_l(chunk, b):
            pltpu.async_copy(pmt_half.at[pl.ds(e_base + chunk * CE, CE)],
                             rows_v.at[b], lsems[b])

        def wait_l(b):
            pltpu.make_async_copy(pmt_half.at[pl.ds(e_base, CE)],
                                  rows_v.at[b], lsems[b]).wait()

        def fire_s(chunk, b):
            for j in range(K):
                seg = rows_v.at[b].at[pl.ds(j * IDXW, IDXW)]
                pltpu.async_copy(seg, m_sh.at[sidx_v.at[chunk * K + j]],
                                 ssems[b], add=True)
                pltpu.async_copy(seg, m_sh.at[didx_v.at[chunk * K + j]],
                                 ssems[b], add=True)

        def wait_s(b):
            for j in range(K):
                seg = rows_v.at[b].at[pl.ds(j * IDXW, IDXW)]
                pltpu.make_async_copy(seg, m_sh.at[sidx_v.at[0]],
                                      ssems[b]).wait()
                pltpu.make_async_copy(seg, m_sh.at[didx_v.at[0]],
                                      ssems[b]).wait()

        fire_l(0, 0)

        def pair(i, carry):
            c0 = 2 * i
            c1 = 2 * i + 1
            wait_l(0)
            fire_s(c0, 0)

            @pl.when(i > 0)
            def _():
                wait_s(1)
            fire_l(c1, 1)
            wait_s(0)

            @pl.when(i < PAIRS - 1)
            def _():
                fire_l(c0 + 2, 0)
            wait_l(1)
            fire_s(c1, 1)
            return carry

        lax.fori_loop(0, PAIRS, pair, 0)
        wait_s(1)
        plsc.subcore_barrier()
        sl = pl.ds(s * ROWS_PT, ROWS_PT)

        @pl.when(c == 0)
        def _():
            pltpu.sync_copy(m_sh.at[sl], out0.at[sl])

        @pl.when(c == 1)
        def _():
            pltpu.sync_copy(m_sh.at[sl], out1.at[sl])

    return pl.kernel(body, out_type=out_type, mesh=_MESH,
                     scratch_types=scratch, compiler_params=_SC_PARAMS)


def _prep(pm, x):
    TJ = 25
    CB = E // (5 * TJ)
    BX = N // 5

    def body(pm_r, x_r, eye_r, pt_r, xs_r):
        j = pl.program_id(1)
        pt = lax.dot_general(pm_r[...], eye_r[...],
                             (((0,), (0,)), ((), ())),
                             preferred_element_type=jnp.float32)
        pt_r[0] = pt[:, :PH].astype(jnp.bfloat16)
        pt_r[1] = pt[:, PH:].astype(jnp.bfloat16)

        @pl.when(j == 0)
        def _():
            xs_r[0] = x_r[:, :DH]
            xs_r[1] = x_r[:, DH:]

    return pl.pallas_call(
        body,
        grid=(5, TJ),
        in_specs=[
            pl.BlockSpec((P, CB), lambda i, j: (0, i * TJ + j)),
            pl.BlockSpec((BX, D), lambda i, j: (i, 0)),
            pl.BlockSpec((P, P), lambda i, j: (0, 0)),
        ],
        out_specs=[
            pl.BlockSpec((2, CB, PH), lambda i, j: (0, i * TJ + j, 0)),
            pl.BlockSpec((2, BX, DH), lambda i, j: (0, i, 0)),
        ],
        out_shape=[
            jax.ShapeDtypeStruct((2, E, PH), jnp.bfloat16),
            jax.ShapeDtypeStruct((2, NPAD, DH), jnp.float32),
        ],
    )(pm, x, jnp.eye(P, dtype=jnp.float32))


def _dense0(a0, a1, cntp, xs, wl, wr, b):
    def body(a0_r, a1_r, c_r, x_r, wl_r, wr_r, b_r, o_r):
        rcp = (1.0 / jnp.maximum(jnp.sum(c_r[...], axis=0), 1.0))[:, None]
        mean = jnp.concatenate((a0_r[...], a1_r[...]), axis=1) * rcp
        xfull = jnp.concatenate((x_r[0], x_r[1]), axis=1)
        h = (jnp.dot(mean, wl_r[...], preferred_element_type=jnp.float32)
             + jnp.dot(xfull, wr_r[...], preferred_element_type=jnp.float32)
             + b_r[...])
        h = jnp.maximum(h, 0.0)
        o_r[0] = h[:, :DH]
        o_r[1] = h[:, DH:]

    return pl.pallas_call(
        body,
        grid=(NPAD // B,),
        in_specs=[
            pl.BlockSpec((B, DH), lambda i: (i, 0)),
            pl.BlockSpec((B, DH), lambda i: (i, 0)),
            pl.BlockSpec((NSUB, B), lambda i: (0, i)),
            pl.BlockSpec((2, B, DH), lambda i: (0, i, 0)),
            pl.BlockSpec((D, D), lambda i: (0, 0)),
            pl.BlockSpec((D, D), lambda i: (0, 0)),
            pl.BlockSpec((1, D), lambda i: (0, 0)),
        ],
        out_specs=pl.BlockSpec((2, B, DH), lambda i: (0, i, 0)),
        out_shape=jax.ShapeDtypeStruct((2, NPAD, DH), jnp.float32),
    )(a0, a1, cntp, xs, wl, wr, b)


def _dense1_final(q0, q1, cntp, h0s, wl, wr, b, m0, m1, wro, bro):
    G = NPAD // B

    def body(q0_r, q1_r, c_r, h0_r, wl_r, wr_r, b_r, m0_r, m1_r, wro_r,
             bro_r, o_r, acc):
        i = pl.program_id(0)
        rcp = (1.0 / jnp.maximum(jnp.sum(c_r[...], axis=0), 1.0))[:, None]
        mean = jnp.concatenate((q0_r[...], q1_r[...]), axis=1) * rcp
        h0full = jnp.concatenate((h0_r[0], h0_r[1]), axis=1)
        h1 = jnp.maximum(
            jnp.dot(mean, wl_r[...], preferred_element_type=jnp.float32)
            + jnp.dot(h0full, wr_r[...], preferred_element_type=jnp.float32)
            + b_r[...], 0.0)
        h1b = h1.astype(jnp.bfloat16)
        pe_top = lax.dot_general(m0_r[...], h1b, (((0,), (0,)), ((), ())),
                                 preferred_element_type=jnp.float32)
        pe_bot = lax.dot_general(m1_r[...], h1b, (((0,), (0,)), ((), ())),
                                 preferred_element_type=jnp.float32)
        pe = jnp.concatenate((pe_top, pe_bot), axis=0)

        @pl.when(i == 0)
        def _():
            acc[...] = pe

        @pl.when(i > 0)
        def _():
            acc[...] += pe

        @pl.when(i == G - 1)
        def _():
            o_r[...] = (jnp.dot(0.5 * acc[...], wro_r[...],
                                preferred_element_type=jnp.float32)
                        + bro_r[...])

    return pl.pallas_call(
        body,
        grid=(G,),
        in_specs=[
            pl.BlockSpec((B, DH), lambda i: (i, 0)),
            pl.BlockSpec((B, DH), lambda i: (i, 0)),
            pl.BlockSpec((NSUB, B), lambda i: (0, i)),
            pl.BlockSpec((2, B, DH), lambda i: (0, i, 0)),
            pl.BlockSpec((D, D), lambda i: (0, 0)),
            pl.BlockSpec((D, D), lambda i: (0, 0)),
            pl.BlockSpec((1, D), lambda i: (0, 0)),
            pl.BlockSpec((B, PH), lambda i: (i, 0)),
            pl.BlockSpec((B, PH), lambda i: (i, 0)),
            pl.BlockSpec((D, D), lambda i: (0, 0)),
            pl.BlockSpec((1, D), lambda i: (0, 0)),
        ],
        out_specs=pl.BlockSpec((P, D), lambda i: (0, 0)),
        out_shape=jax.ShapeDtypeStruct((P, D), jnp.float32),
        scratch_shapes=[pltpu.VMEM((P, D), jnp.float32)],
    )(q0, q1, cntp, h0s, wl, wr, b, m0, m1, wro, bro)


def kernel(node_features, edge_index, path_masks,
           W_l0, W_r0, b0, W_l1, W_r1, b1, W_ro, b_ro):
    sidx3d = edge_index[0].reshape(NSUB, RPT, IDXW)
    didx3d = edge_index[1].reshape(NSUB, RPT, IDXW)
    zrows_d = jnp.zeros((ROWS_PT, DH), jnp.float32)
    zrows_p = jnp.zeros((ROWS_PT, PH), jnp.bfloat16)
    b0r = b0.reshape(1, D)
    b1r = b1.reshape(1, D)
    bror = b_ro.reshape(1, D)

    pmt, xs = _prep(path_masks, node_features)
    a0, a1, cntp = _agg_call(True)(xs, sidx3d, didx3d, zrows_d)
    h0s = _dense0(a0, a1, cntp, xs, W_l0, W_r0, b0r)
    q0, q1 = _agg_call(False)(h0s, sidx3d, didx3d, zrows_d)
    m0, m1 = _mbuild_call()(pmt, sidx3d, didx3d, zrows_p)
    out = _dense1_final(q0, q1, cntp, h0s, W_l1, W_r1, b1r, m0, m1,
                        W_ro, bror)
    return out.reshape(-1)


# speedup vs baseline: 1.0901x; 1.0901x over previous
import jax
import jax.numpy as jnp
from jax import lax
from jax.experimental import pallas as pl
from jax.experimental.pallas import tpu as pltpu
from jax.experimental.pallas import tpu_sc as plsc

N = 10000
NPAD = 10240
E = 320000
D = 128
DH = D // 2
P = 64
PH = P // 2

NCORES = 2
NSUB = 16
NTILES = NCORES * NSUB

IDXW = 80
NROWS = E // IDXW
RPT = NROWS // NSUB
K = 5
CHUNKS = RPT // K
PAIRS = CHUNKS // 2
CE = K * IDXW
ROWS_PT = NPAD // NSUB
B = 2048

_MESH = plsc.VectorSubcoreMesh(core_axis_name="c", subcore_axis_name="s")
_SC_PARAMS = pltpu.CompilerParams(needs_layout_passes=False,
                                  use_tc_tiling_on_sc=False)


def _agg_call(with_cnt):
    out_type = [
        jax.ShapeDtypeStruct((NPAD, DH), jnp.float32),
        jax.ShapeDtypeStruct((NPAD, DH), jnp.float32),
    ]
    if with_cnt:
        out_type.append(jax.ShapeDtypeStruct((NSUB, NPAD), jnp.float32))
    scratch = [
        pltpu.VMEM((2, K, IDXW), jnp.int32),
        pltpu.VMEM((2, K, IDXW), jnp.int32),
        pltpu.VMEM((2, K, IDXW, DH), jnp.float32),
        pltpu.VMEM_SHARED((NPAD, DH), jnp.float32),
        pltpu.SemaphoreType.DMA,
        pltpu.SemaphoreType.DMA,
        pltpu.SemaphoreType.DMA,
        pltpu.SemaphoreType.DMA,
        pltpu.SemaphoreType.DMA,
        pltpu.SemaphoreType.DMA,
    ]
    if with_cnt:
        scratch.append(pltpu.VMEM((NPAD,), jnp.float32))

    def body(h_hbm, sidx_hbm, didx_hbm, zrows_hbm, *rest):
        if with_cnt:
            (out0, out1, out_cnt, sidx_v, didx_v, rows_v, acc_sh,
             isem0, isem1, gsem0, gsem1, ssem0, ssem1, cnt_v) = rest
        else:
            (out0, out1, sidx_v, didx_v, rows_v, acc_sh,
             isem0, isem1, gsem0, gsem1, ssem0, ssem1) = rest
        c = lax.axis_index("c")
        s = lax.axis_index("s")
        pltpu.sync_copy(zrows_hbm, acc_sh.at[pl.ds(s * ROWS_PT, ROWS_PT)])
        if with_cnt:
            def zero_cnt(i, carry):
                cnt_v[pl.ds(i * 16, 16)] = jnp.zeros((16,), jnp.float32)
                return carry
            lax.fori_loop(0, NPAD // 16, zero_cnt, 0)
        plsc.subcore_barrier()
        h_half = h_hbm.at[c]
        s_rows = sidx_hbm.at[s]
        d_rows = didx_hbm.at[s]
        isems = (isem0, isem1)
        gsems = (gsem0, gsem1)
        ssems = (ssem0, ssem1)

        def fire_i(chunk, b):
            pltpu.async_copy(s_rows.at[pl.ds(chunk * K, K)], sidx_v.at[b],
                             isems[b])
            pltpu.async_copy(d_rows.at[pl.ds(chunk * K, K)], didx_v.at[b],
                             isems[b])

        def wait_i(b):
            pltpu.make_async_copy(s_rows.at[pl.ds(0, K)], sidx_v.at[b],
                                  isems[b]).wait()
            pltpu.make_async_copy(d_rows.at[pl.ds(0, K)], didx_v.at[b],
                                  isems[b]).wait()

        def fire_g(b):
            for j in range(K):
                pltpu.async_copy(h_half.at[sidx_v.at[b, j]],
                                 rows_v.at[b, j], gsems[b])

        def wait_g(b):
            for j in range(K):
                pltpu.make_async_copy(h_half.at[sidx_v.at[b, 0]],
                                      rows_v.at[b, j], gsems[b]).wait()

        def fire_s(b):
            for j in range(K):
                pltpu.async_copy(rows_v.at[b, j],
                                 acc_sh.at[didx_v.at[b, j]],
                                 ssems[b], add=True)

        def wait_s(b):
            for j in range(K):
                pltpu.make_async_copy(rows_v.at[b, j],
                                      acc_sh.at[didx_v.at[b, 0]],
                                      ssems[b]).wait()

        def do_cnt(b):
            if with_cnt:
                ones16 = jnp.ones((16,), jnp.float32)
                for j in range(K):
                    for jj in range(IDXW // 16):
                        idx16 = didx_v[b, j, pl.ds(jj * 16, 16)]
                        plsc.addupdate_scatter(cnt_v, [idx16], ones16)

        fire_i(0, 0)

        def pair(i, carry):
            c0 = 2 * i
            c1 = 2 * i + 1

            @pl.when(i > 0)
            def _():
                wait_s(1)
            fire_i(c1, 1)
            wait_i(0)
            fire_g(0)
            do_cnt(0)
            wait_g(0)
            fire_s(0)
            wait_i(1)
            fire_g(1)
            do_cnt(1)
            wait_s(0)

            @pl.when(i < PAIRS - 1)
            def _():
                fire_i(c0 + 2, 0)
            wait_g(1)
            fire_s(1)
            return carry

        lax.fori_loop(0, PAIRS, pair, 0)
        wait_s(1)

        if with_cnt:
            @pl.when(c == 0)
            def _():
                pltpu.sync_copy(cnt_v, out_cnt.at[s])

        plsc.subcore_barrier()
        sl = pl.ds(s * ROWS_PT, ROWS_PT)

        @pl.when(c == 0)
        def _():
            pltpu.sync_copy(acc_sh.at[sl], out0.at[sl])

        @pl.when(c == 1)
        def _():
            pltpu.sync_copy(acc_sh.at[sl], out1.at[sl])

    return pl.kernel(body, out_type=out_type, mesh=_MESH,
                     scratch_types=scratch, compiler_params=_SC_PARAMS)


def _mbuild_call():
    out_type = [
        jax.ShapeDtypeStruct((NPAD, PH), jnp.float32),
        jax.ShapeDtypeStruct((NPAD, PH), jnp.float32),
    ]
    scratch = [
        pltpu.VMEM((RPT, IDXW), jnp.int32),
        pltpu.VMEM((RPT, IDXW), jnp.int32),
        pltpu.VMEM((2, CE, PH), jnp.float32),
        pltpu.VMEM_SHARED((NPAD, PH), jnp.float32),
        pltpu.SemaphoreType.DMA,
        pltpu.SemaphoreType.DMA,
        pltpu.SemaphoreType.DMA,
        pltpu.SemaphoreType.DMA,
    ]

    def body(pmt_hbm, sidx_hbm, didx_hbm, zrows_hbm,
             out0, out1, sidx_v, didx_v, rows_v, m_sh,
             lsem0, lsem1, ssem0, ssem1):
        c = lax.axis_index("c")
        s = lax.axis_index("s")
        pltpu.sync_copy(zrows_hbm, m_sh.at[pl.ds(s * ROWS_PT, ROWS_PT)])
        pltpu.sync_copy(sidx_hbm.at[s], sidx_v)
        pltpu.sync_copy(didx_hbm.at[s], didx_v)
        plsc.subcore_barrier()
        pmt_half = pmt_hbm.at[c]
        e_base = s * (E // NSUB)
        lsems = (lsem0, lsem1)
        ssems = (ssem0, ssem1)

        def fire_l(chunk, b):
            pltpu.async_copy(pmt_half.at[pl.ds(e_base + chunk * CE, CE)],
                             rows_v.at[b], lsems[b])

        def wait_l(b):
            pltpu.make_async_copy(pmt_half.at[pl.ds(e_base, CE)],
                                  rows_v.at[b], lsems[b]).wait()

        def fire_s(chunk, b):
            for j in range(K):
                seg = rows_v.at[b].at[pl.ds(j * IDXW, IDXW)]
                pltpu.async_copy(seg, m_sh.at[sidx_v.at[chunk * K + j]],
                                 ssems[b], add=True)
                pltpu.async_copy(seg, m_sh.at[didx_v.at[chunk * K + j]],
                                 ssems[b], add=True)

        def wait_s(b):
            for j in range(K):
                seg = rows_v.at[b].at[pl.ds(j * IDXW, IDXW)]
                pltpu.make_async_copy(seg, m_sh.at[sidx_v.at[0]],
                                      ssems[b]).wait()
                pltpu.make_async_copy(seg, m_sh.at[didx_v.at[0]],
                                      ssems[b]).wait()

        fire_l(0, 0)

        def pair(i, carry):
            c0 = 2 * i
            c1 = 2 * i + 1
            wait_l(0)
            fire_s(c0, 0)

            @pl.when(i > 0)
            def _():
                wait_s(1)
            fire_l(c1, 1)
            wait_s(0)

            @pl.when(i < PAIRS - 1)
            def _():
                fire_l(c0 + 2, 0)
            wait_l(1)
            fire_s(c1, 1)
            return carry

        lax.fori_loop(0, PAIRS, pair, 0)
        wait_s(1)
        plsc.subcore_barrier()
        sl = pl.ds(s * ROWS_PT, ROWS_PT)

        @pl.when(c == 0)
        def _():
            pltpu.sync_copy(m_sh.at[sl], out0.at[sl])

        @pl.when(c == 1)
        def _():
            pltpu.sync_copy(m_sh.at[sl], out1.at[sl])

    return pl.kernel(body, out_type=out_type, mesh=_MESH,
                     scratch_types=scratch, compiler_params=_SC_PARAMS)


def _transpose_pm(pm):
    cb = 2560

    def body(in_ref, out_ref):
        out_ref[0] = in_ref[:PH, :].T
        out_ref[1] = in_ref[PH:, :].T

    return pl.pallas_call(
        body,
        grid=(E // cb,),
        in_specs=[pl.BlockSpec((P, cb), lambda i: (0, i))],
        out_specs=pl.BlockSpec((2, cb, PH), lambda i: (0, i, 0)),
        out_shape=jax.ShapeDtypeStruct((2, E, PH), jnp.float32),
    )(pm)


def _dense0(a0, a1, cntp, x, wl, wr, b):
    def body(a0_r, a1_r, c_r, x_r, wl_r, wr_r, b_r, o_r):
        rcp = (1.0 / jnp.maximum(jnp.sum(c_r[...], axis=0), 1.0))[:, None]
        mean = jnp.concatenate((a0_r[...], a1_r[...]), axis=1) * rcp
        h = (jnp.dot(mean, wl_r[...], preferred_element_type=jnp.float32)
             + jnp.dot(x_r[...], wr_r[...], preferred_element_type=jnp.float32)
             + b_r[...])
        h = jnp.maximum(h, 0.0)
        o_r[0] = h[:, :DH]
        o_r[1] = h[:, DH:]

    return pl.pallas_call(
        body,
        grid=(NPAD // B,),
        in_specs=[
            pl.BlockSpec((B, DH), lambda i: (i, 0)),
            pl.BlockSpec((B, DH), lambda i: (i, 0)),
            pl.BlockSpec((NSUB, B), lambda i: (0, i)),
            pl.BlockSpec((B, D), lambda i: (i, 0)),
            pl.BlockSpec((D, D), lambda i: (0, 0)),
            pl.BlockSpec((D, D), lambda i: (0, 0)),
            pl.BlockSpec((1, D), lambda i: (0, 0)),
        ],
        out_specs=pl.BlockSpec((2, B, DH), lambda i: (0, i, 0)),
        out_shape=jax.ShapeDtypeStruct((2, NPAD, DH), jnp.float32),
    )(a0, a1, cntp, x, wl, wr, b)


def _dense1_final(q0, q1, cntp, h0s, wl, wr, b, m0, m1, wro, bro):
    G = NPAD // B

    def body(q0_r, q1_r, c_r, h0_r, wl_r, wr_r, b_r, m0_r, m1_r, wro_r,
             bro_r, o_r, acc):
        i = pl.program_id(0)
        rcp = (1.0 / jnp.maximum(jnp.sum(c_r[...], axis=0), 1.0))[:, None]
        mean = jnp.concatenate((q0_r[...], q1_r[...]), axis=1) * rcp
        h0full = jnp.concatenate((h0_r[0], h0_r[1]), axis=1)
        h1 = jnp.maximum(
            jnp.dot(mean, wl_r[...], preferred_element_type=jnp.float32)
            + jnp.dot(h0full, wr_r[...], preferred_element_type=jnp.float32)
            + b_r[...], 0.0)
        pe_top = lax.dot_general(m0_r[...], h1, (((0,), (0,)), ((), ())),
                                 preferred_element_type=jnp.float32)
        pe_bot = lax.dot_general(m1_r[...], h1, (((0,), (0,)), ((), ())),
                                 preferred_element_type=jnp.float32)
        pe = jnp.concatenate((pe_top, pe_bot), axis=0)

        @pl.when(i == 0)
        def _():
            acc[...] = pe

        @pl.when(i > 0)
        def _():
            acc[...] += pe

        @pl.when(i == G - 1)
        def _():
            o_r[...] = (jnp.dot(0.5 * acc[...], wro_r[...],
                                preferred_element_type=jnp.float32)
                        + bro_r[...])

    return pl.pallas_call(
        body,
        grid=(G,),
        in_specs=[
            pl.BlockSpec((B, DH), lambda i: (i, 0)),
            pl.BlockSpec((B, DH), lambda i: (i, 0)),
            pl.BlockSpec((NSUB, B), lambda i: (0, i)),
            pl.BlockSpec((2, B, DH), lambda i: (0, i, 0)),
            pl.BlockSpec((D, D), lambda i: (0, 0)),
            pl.BlockSpec((D, D), lambda i: (0, 0)),
            pl.BlockSpec((1, D), lambda i: (0, 0)),
            pl.BlockSpec((B, PH), lambda i: (i, 0)),
            pl.BlockSpec((B, PH), lambda i: (i, 0)),
            pl.BlockSpec((D, D), lambda i: (0, 0)),
            pl.BlockSpec((1, D), lambda i: (0, 0)),
        ],
        out_specs=pl.BlockSpec((P, D), lambda i: (0, 0)),
        out_shape=jax.ShapeDtypeStruct((P, D), jnp.float32),
        scratch_shapes=[pltpu.VMEM((P, D), jnp.float32)],
    )(q0, q1, cntp, h0s, wl, wr, b, m0, m1, wro, bro)


def kernel(node_features, edge_index, path_masks,
           W_l0, W_r0, b0, W_l1, W_r1, b1, W_ro, b_ro):
    sidx3d = edge_index[0].reshape(NSUB, RPT, IDXW)
    didx3d = edge_index[1].reshape(NSUB, RPT, IDXW)
    zrows_d = jnp.zeros((ROWS_PT, DH), jnp.float32)
    zrows_p = jnp.zeros((ROWS_PT, PH), jnp.float32)
    b0r = b0.reshape(1, D)
    b1r = b1.reshape(1, D)
    bror = b_ro.reshape(1, D)
    xs = jnp.stack((node_features[:, :DH], node_features[:, DH:]))
    xpad = jnp.pad(node_features, ((0, NPAD - N), (0, 0)))

    pmt = _transpose_pm(path_masks)
    a0, a1, cntp = _agg_call(True)(xs, sidx3d, didx3d, zrows_d)
    h0s = _dense0(a0, a1, cntp, xpad, W_l0, W_r0, b0r)
    q0, q1 = _agg_call(False)(h0s, sidx3d, didx3d, zrows_d)
    m0, m1 = _mbuild_call()(pmt, sidx3d, didx3d, zrows_p)
    out = _dense1_final(q0, q1, cntp, h0s, W_l1, W_r1, b1r, m0, m1,
                        W_ro, bror)
    return out.reshape(-1)


# speedup vs baseline: 1.0912x; 1.0011x over previous
import jax
import jax.numpy as jnp
from jax import lax
from jax.experimental import pallas as pl
from jax.experimental.pallas import tpu as pltpu
from jax.experimental.pallas import tpu_sc as plsc

N = 10000
NPAD = 10240
E = 320000
D = 128
DH = D // 2
P = 64
PH = P // 2

NCORES = 2
NSUB = 16
NTILES = NCORES * NSUB

IDXW = 80
NROWS = E // IDXW
RPT = NROWS // NSUB
K = 5
CHUNKS = RPT // K
PAIRS = CHUNKS // 2
CE = K * IDXW
ROWS_PT = NPAD // NSUB
B = 2048

_MESH = plsc.VectorSubcoreMesh(core_axis_name="c", subcore_axis_name="s")
_SC_PARAMS = pltpu.CompilerParams(needs_layout_passes=False,
                                  use_tc_tiling_on_sc=False)


def _agg_call(with_cnt):
    out_type = [
        jax.ShapeDtypeStruct((NPAD, DH), jnp.float32),
        jax.ShapeDtypeStruct((NPAD, DH), jnp.float32),
    ]
    if with_cnt:
        out_type.append(jax.ShapeDtypeStruct((NSUB, NPAD), jnp.float32))
    scratch = [
        pltpu.VMEM((2, K, IDXW), jnp.int32),
        pltpu.VMEM((2, K, IDXW), jnp.int32),
        pltpu.VMEM((2, K, IDXW, DH), jnp.float32),
        pltpu.VMEM_SHARED((NPAD, DH), jnp.float32),
        pltpu.SemaphoreType.DMA,
        pltpu.SemaphoreType.DMA,
        pltpu.SemaphoreType.DMA,
        pltpu.SemaphoreType.DMA,
        pltpu.SemaphoreType.DMA,
        pltpu.SemaphoreType.DMA,
    ]
    if with_cnt:
        scratch.append(pltpu.VMEM((NPAD,), jnp.float32))

    def body(h_hbm, sidx_hbm, didx_hbm, zrows_hbm, *rest):
        if with_cnt:
            (out0, out1, out_cnt, sidx_v, didx_v, rows_v, acc_sh,
             isem0, isem1, gsem0, gsem1, ssem0, ssem1, cnt_v) = rest
        else:
            (out0, out1, sidx_v, didx_v, rows_v, acc_sh,
             isem0, isem1, gsem0, gsem1, ssem0, ssem1) = rest
        c = lax.axis_index("c")
        s = lax.axis_index("s")
        pltpu.sync_copy(zrows_hbm, acc_sh.at[pl.ds(s * ROWS_PT, ROWS_PT)])
        if with_cnt:
            def zero_cnt(i, carry):
                cnt_v[pl.ds(i * 16, 16)] = jnp.zeros((16,), jnp.float32)
                return carry
            lax.fori_loop(0, NPAD // 16, zero_cnt, 0)
        plsc.subcore_barrier()
        h_half = h_hbm.at[c]
        s_rows = sidx_hbm.at[s]
        d_rows = didx_hbm.at[s]
        isems = (isem0, isem1)
        gsems = (gsem0, gsem1)
        ssems = (ssem0, ssem1)

        def fire_i(chunk, b):
            pltpu.async_copy(s_rows.at[pl.ds(chunk * K, K)], sidx_v.at[b],
                             isems[b])
            pltpu.async_copy(d_rows.at[pl.ds(chunk * K, K)], didx_v.at[b],
                             isems[b])

        def wait_i(b):
            pltpu.make_async_copy(s_rows.at[pl.ds(0, K)], sidx_v.at[b],
                                  isems[b]).wait()
            pltpu.make_async_copy(d_rows.at[pl.ds(0, K)], didx_v.at[b],
                                  isems[b]).wait()

        def fire_g(b):
            for j in range(K):
                pltpu.async_copy(h_half.at[sidx_v.at[b, j]],
                                 rows_v.at[b, j], gsems[b])

        def wait_g(b):
            for j in range(K):
                pltpu.make_async_copy(h_half.at[sidx_v.at[b, 0]],
                                      rows_v.at[b, j], gsems[b]).wait()

        def fire_s(b):
            for j in range(K):
                pltpu.async_copy(rows_v.at[b, j],
                                 acc_sh.at[didx_v.at[b, j]],
                                 ssems[b], add=True)

        def wait_s(b):
            for j in range(K):
                pltpu.make_async_copy(rows_v.at[b, j],
                                      acc_sh.at[didx_v.at[b, 0]],
                                      ssems[b]).wait()

        def do_cnt(b):
            if with_cnt:
                ones16 = jnp.ones((16,), jnp.float32)
                for j in range(K):
                    for jj in range(IDXW // 16):
                        idx16 = didx_v[b, j, pl.ds(jj * 16, 16)]
                        plsc.addupdate_scatter(cnt_v, [idx16], ones16)

        fire_i(0, 0)

        def pair(i, carry):
            c0 = 2 * i
            c1 = 2 * i + 1

            @pl.when(i > 0)
            def _():
                wait_s(1)
            fire_i(c1, 1)
            wait_i(0)
            fire_g(0)
            do_cnt(0)
            wait_g(0)
            fire_s(0)
            wait_i(1)
            fire_g(1)
            do_cnt(1)
            wait_s(0)

            @pl.when(i < PAIRS - 1)
            def _():
                fire_i(c0 + 2, 0)
            wait_g(1)
            fire_s(1)
            return carry

        lax.fori_loop(0, PAIRS, pair, 0)
        wait_s(1)

        if with_cnt:
            @pl.when(c == 0)
            def _():
                pltpu.sync_copy(cnt_v, out_cnt.at[s])

        plsc.subcore_barrier()
        sl = pl.ds(s * ROWS_PT, ROWS_PT)

        @pl.when(c == 0)
        def _():
            pltpu.sync_copy(acc_sh.at[sl], out0.at[sl])

        @pl.when(c == 1)
        def _():
            pltpu.sync_copy(acc_sh.at[sl], out1.at[sl])

    return pl.kernel(body, out_type=out_type, mesh=_MESH,
                     scratch_types=scratch, compiler_params=_SC_PARAMS)


def _mbuild_call():
    out_type = [
        jax.ShapeDtypeStruct((NPAD, PH), jnp.float32),
        jax.ShapeDtypeStruct((NPAD, PH), jnp.float32),
    ]
    scratch = [
        pltpu.VMEM((RPT, IDXW), jnp.int32),
        pltpu.VMEM((RPT, IDXW), jnp.int32),
        pltpu.VMEM((2, CE, PH), jnp.float32),
        pltpu.VMEM_SHARED((NPAD, PH), jnp.float32),
        pltpu.SemaphoreType.DMA,
        pltpu.SemaphoreType.DMA,
        pltpu.SemaphoreType.DMA,
        pltpu.SemaphoreType.DMA,
    ]

    def body(pmt_hbm, sidx_hbm, didx_hbm, zrows_hbm,
             out0, out1, sidx_v, didx_v, rows_v, m_sh,
             lsem0, lsem1, ssem0, ssem1):
        c = lax.axis_index("c")
        s = lax.axis_index("s")
        pltpu.sync_copy(zrows_hbm, m_sh.at[pl.ds(s * ROWS_PT, ROWS_PT)])
        pltpu.sync_copy(sidx_hbm.at[s], sidx_v)
        pltpu.sync_copy(didx_hbm.at[s], didx_v)
        plsc.subcore_barrier()
        pmt_half = pmt_hbm.at[c]
        e_base = s * (E // NSUB)
        lsems = (lsem0, lsem1)
        ssems = (ssem0, ssem1)

        def fire_l(chunk, b):
            pltpu.async_copy(pmt_half.at[pl.ds(e_base + chunk * CE, CE)],
                             rows_v.at[b], lsems[b])

        def wait_l(b):
            pltpu.make_async_copy(pmt_half.at[pl.ds(e_base, CE)],
                                  rows_v.at[b], lsems[b]).wait()

        def fire_s(chunk, b):
            for j in range(K):
                seg = rows_v.at[b].at[pl.ds(j * IDXW, IDXW)]
                pltpu.async_copy(seg, m_sh.at[sidx_v.at[chunk * K + j]],
                                 ssems[b], add=True)
                pltpu.async_copy(seg, m_sh.at[didx_v.at[chunk * K + j]],
                                 ssems[b], add=True)

        def wait_s(b):
            for j in range(K):
                seg = rows_v.at[b].at[pl.ds(j * IDXW, IDXW)]
                pltpu.make_async_copy(seg, m_sh.at[sidx_v.at[0]],
                                      ssems[b]).wait()
                pltpu.make_async_copy(seg, m_sh.at[didx_v.at[0]],
                                      ssems[b]).wait()

        fire_l(0, 0)

        def pair(i, carry):
            c0 = 2 * i
            c1 = 2 * i + 1
            wait_l(0)
            fire_s(c0, 0)

            @pl.when(i > 0)
            def _():
                wait_s(1)
            fire_l(c1, 1)
            wait_s(0)

            @pl.when(i < PAIRS - 1)
            def _():
                fire_l(c0 + 2, 0)
            wait_l(1)
            fire_s(c1, 1)
            return carry

        lax.fori_loop(0, PAIRS, pair, 0)
        wait_s(1)
        plsc.subcore_barrier()
        sl = pl.ds(s * ROWS_PT, ROWS_PT)

        @pl.when(c == 0)
        def _():
            pltpu.sync_copy(m_sh.at[sl], out0.at[sl])

        @pl.when(c == 1)
        def _():
            pltpu.sync_copy(m_sh.at[sl], out1.at[sl])

    return pl.kernel(body, out_type=out_type, mesh=_MESH,
                     scratch_types=scratch, compiler_params=_SC_PARAMS)


def _transpose_pm(pm):
    cb = 2560

    def body(in_ref, out_ref):
        out_ref[0] = in_ref[:PH, :].T
        out_ref[1] = in_ref[PH:, :].T

    return pl.pallas_call(
        body,
        grid=(E // cb,),
        in_specs=[pl.BlockSpec((P, cb), lambda i: (0, i))],
        out_specs=pl.BlockSpec((2, cb, PH), lambda i: (0, i, 0)),
        out_shape=jax.ShapeDtypeStruct((2, E, PH), jnp.float32),
    )(pm)


def _dense0(a0, a1, cntp, x, wl, wr, b):
    def body(a0_r, a1_r, c_r, x_r, wl_r, wr_r, b_r, o_r):
        rcp = (1.0 / jnp.maximum(jnp.sum(c_r[...], axis=0), 1.0))[:, None]
        mean = jnp.concatenate((a0_r[...], a1_r[...]), axis=1) * rcp
        h = (jnp.dot(mean, wl_r[...], preferred_element_type=jnp.float32)
             + jnp.dot(x_r[...], wr_r[...], preferred_element_type=jnp.float32)
             + b_r[...])
        h = jnp.maximum(h, 0.0)
        o_r[0] = h[:, :DH]
        o_r[1] = h[:, DH:]

    return pl.pallas_call(
        body,
        grid=(NPAD // B,),
        in_specs=[
            pl.BlockSpec((B, DH), lambda i: (i, 0)),
            pl.BlockSpec((B, DH), lambda i: (i, 0)),
            pl.BlockSpec((NSUB, B), lambda i: (0, i)),
            pl.BlockSpec((B, D), lambda i: (i, 0)),
            pl.BlockSpec((D, D), lambda i: (0, 0)),
            pl.BlockSpec((D, D), lambda i: (0, 0)),
            pl.BlockSpec((1, D), lambda i: (0, 0)),
        ],
        out_specs=pl.BlockSpec((2, B, DH), lambda i: (0, i, 0)),
        out_shape=jax.ShapeDtypeStruct((2, NPAD, DH), jnp.float32),
    )(a0, a1, cntp, x, wl, wr, b)


def _dense1_final(q0, q1, cntp, h0s, wl, wr, b, m0, m1, wro, bro):
    G = NPAD // B

    def body(q0_r, q1_r, c_r, h0_r, wl_r, wr_r, b_r, m0_r, m1_r, wro_r,
             bro_r, o_r, acc):
        i = pl.program_id(0)
        rcp = (1.0 / jnp.maximum(jnp.sum(c_r[...], axis=0), 1.0))[:, None]
        mean = jnp.concatenate((q0_r[...], q1_r[...]), axis=1) * rcp
        h0full = jnp.concatenate((h0_r[0], h0_r[1]), axis=1)
        h1 = jnp.maximum(
            jnp.dot(mean, wl_r[...], preferred_element_type=jnp.float32)
            + jnp.dot(h0full, wr_r[...], preferred_element_type=jnp.float32)
            + b_r[...], 0.0)
        pe_top = lax.dot_general(m0_r[...], h1, (((0,), (0,)), ((), ())),
                                 preferred_element_type=jnp.float32)
        pe_bot = lax.dot_general(m1_r[...], h1, (((0,), (0,)), ((), ())),
                                 preferred_element_type=jnp.float32)
        pe = jnp.concatenate((pe_top, pe_bot), axis=0)

        @pl.when(i == 0)
        def _():
            acc[...] = pe

        @pl.when(i > 0)
        def _():
            acc[...] += pe

        @pl.when(i == G - 1)
        def _():
            o_r[...] = (jnp.dot(0.5 * acc[...], wro_r[...],
                                preferred_element_type=jnp.float32)
                        + bro_r[...])

    return pl.pallas_call(
        body,
        grid=(G,),
        in_specs=[
            pl.BlockSpec((B, DH), lambda i: (i, 0)),
            pl.BlockSpec((B, DH), lambda i: (i, 0)),
            pl.BlockSpec((NSUB, B), lambda i: (0, i)),
            pl.BlockSpec((2, B, DH), lambda i: (0, i, 0)),
            pl.BlockSpec((D, D), lambda i: (0, 0)),
            pl.BlockSpec((D, D), lambda i: (0, 0)),
            pl.BlockSpec((1, D), lambda i: (0, 0)),
            pl.BlockSpec((B, PH), lambda i: (i, 0)),
            pl.BlockSpec((B, PH), lambda i: (i, 0)),
            pl.BlockSpec((D, D), lambda i: (0, 0)),
            pl.BlockSpec((1, D), lambda i: (0, 0)),
        ],
        out_specs=pl.BlockSpec((P, D), lambda i: (0, 0)),
        out_shape=jax.ShapeDtypeStruct((P, D), jnp.float32),
        scratch_shapes=[pltpu.VMEM((P, D), jnp.float32)],
    )(q0, q1, cntp, h0s, wl, wr, b, m0, m1, wro, bro)


def kernel(node_features, edge_index, path_masks,
           W_l0, W_r0, b0, W_l1, W_r1, b1, W_ro, b_ro):
    sidx3d = edge_index[0].reshape(NSUB, RPT, IDXW)
    didx3d = edge_index[1].reshape(NSUB, RPT, IDXW)
    zrows_d = jnp.zeros((ROWS_PT, DH), jnp.float32)
    zrows_p = jnp.zeros((ROWS_PT, PH), jnp.float32)
    b0r = b0.reshape(1, D)
    b1r = b1.reshape(1, D)
    bror = b_ro.reshape(1, D)
    xs = jnp.stack((node_features[:, :DH], node_features[:, DH:]))
    xpad = jnp.pad(node_features, ((0, NPAD - N), (0, 0)))

    pmt = _transpose_pm(path_masks)
    a0, a1, cntp = _agg_call(True)(xs, sidx3d, didx3d, zrows_d)
    m0, m1 = _mbuild_call()(pmt, sidx3d, didx3d, zrows_p)
    h0s = _dense0(a0, a1, cntp, xpad, W_l0, W_r0, b0r)
    q0, q1 = _agg_call(False)(h0s, sidx3d, didx3d, zrows_d)
    out = _dense1_final(q0, q1, cntp, h0s, W_l1, W_r1, b1r, m0, m1,
                        W_ro, bror)
    return out.reshape(-1)
